# baseline (speedup 1.0000x reference)
import functools

import jax
import jax.numpy as jnp
from jax.experimental import pallas as pl
from jax.experimental.pallas import tpu as pltpu

_BM = 512
_NBUF = 4


def _router_body(x_hbm, w_ref, o_hbm, xbuf, obuf, in_sems, out_sems,
                 *, nsteps):
    nmacro = nsteps // _NBUF

    def in_copy(s, j):
        return pltpu.make_async_copy(
            x_hbm.at[pl.ds(s * _BM, _BM), :], xbuf.at[j], in_sems.at[j])

    def out_copy(s, j):
        return pltpu.make_async_copy(
            obuf.at[j], o_hbm.at[pl.ds(s * _BM, _BM), :], out_sems.at[j])

    for j in range(_NBUF):
        in_copy(j, j).start()

    def macro(m, carry):
        for j in range(_NBUF):
            s = m * _NBUF + j
            in_copy(s, j).wait()

            @pl.when(m >= 1)
            def _():
                out_copy(s - _NBUF, j).wait()

            obuf[j] = jnp.dot(xbuf[j], w_ref[...],
                              preferred_element_type=jnp.float32)
            out_copy(s, j).start()

            @pl.when(s + _NBUF < nsteps)
            def _():
                in_copy(s + _NBUF, j).start()

        return carry

    jax.lax.fori_loop(0, nmacro, macro, 0)

    for j in range(_NBUF):
        out_copy(nsteps - _NBUF + j, j).wait()


def kernel(hidden_states, W):
    M, K = hidden_states.shape
    K2, N = W.shape
    assert K == K2 and M % (_BM * _NBUF) == 0
    nsteps = M // _BM
    return pl.pallas_call(
        functools.partial(_router_body, nsteps=nsteps),
        in_specs=[
            pl.BlockSpec(memory_space=pl.ANY),
            pl.BlockSpec((K, N), lambda: (0, 0)),
        ],
        out_specs=pl.BlockSpec(memory_space=pl.ANY),
        out_shape=jax.ShapeDtypeStruct((M, N), jnp.float32),
        scratch_shapes=[
            pltpu.VMEM((_NBUF, _BM, K), jnp.float32),
            pltpu.VMEM((_NBUF, _BM, N), jnp.float32),
            pltpu.SemaphoreType.DMA((_NBUF,)),
            pltpu.SemaphoreType.DMA((_NBUF,)),
        ],
    )(hidden_states, W)


# speedup vs baseline: 1.0004x; 1.0004x over previous
import functools

import jax
import jax.numpy as jnp
from jax.experimental import pallas as pl
from jax.experimental.pallas import tpu as pltpu

_BM = 512
_NBUF = 4


def _router_body(x_hbm, w_ref, o_hbm, xbuf, obuf, in_sems, out_sems,
                 *, nsteps):
    nmacro = nsteps // _NBUF

    _NSPLIT = 8
    _SUB = _BM // _NSPLIT

    def in_copies(s, j):
        return [pltpu.make_async_copy(
            x_hbm.at[pl.ds(s * _BM + p * _SUB, _SUB), :],
            xbuf.at[j, pl.ds(p * _SUB, _SUB), :],
            in_sems.at[j]) for p in range(_NSPLIT)]

    def out_copy(s, j):
        return pltpu.make_async_copy(
            obuf.at[j], o_hbm.at[pl.ds(s * _BM, _BM), :], out_sems.at[j])

    for j in range(_NBUF):
        for c in in_copies(j, j):
            c.start()

    def macro(m, carry):
        for j in range(_NBUF):
            s = m * _NBUF + j
            for c in in_copies(s, j):
                c.wait()

            @pl.when(m >= 1)
            def _():
                out_copy(s - _NBUF, j).wait()

            obuf[j] = jnp.dot(xbuf[j], w_ref[...],
                              preferred_element_type=jnp.float32)
            out_copy(s, j).start()

            @pl.when(s + _NBUF < nsteps)
            def _():
                for c in in_copies(s + _NBUF, j):
                    c.start()

        return carry

    jax.lax.fori_loop(0, nmacro, macro, 0)

    for j in range(_NBUF):
        out_copy(nsteps - _NBUF + j, j).wait()


def kernel(hidden_states, W):
    M, K = hidden_states.shape
    K2, N = W.shape
    assert K == K2 and M % (_BM * _NBUF) == 0
    nsteps = M // _BM
    return pl.pallas_call(
        functools.partial(_router_body, nsteps=nsteps),
        in_specs=[
            pl.BlockSpec(memory_space=pl.ANY),
            pl.BlockSpec((K, N), lambda: (0, 0)),
        ],
        out_specs=pl.BlockSpec(memory_space=pl.ANY),
        out_shape=jax.ShapeDtypeStruct((M, N), jnp.float32),
        scratch_shapes=[
            pltpu.VMEM((_NBUF, _BM, K), jnp.float32),
            pltpu.VMEM((_NBUF, _BM, N), jnp.float32),
            pltpu.SemaphoreType.DMA((_NBUF,)),
            pltpu.SemaphoreType.DMA((_NBUF,)),
        ],
    )(hidden_states, W)


# speedup vs baseline: 1.0450x; 1.0445x over previous
import jax
import jax.numpy as jnp
from jax.experimental import pallas as pl
from jax.experimental.pallas import tpu as pltpu

_BM = 512


def _router_matmul_kernel(x_ref, w_ref, o_ref):
    o_ref[...] = jnp.dot(x_ref[...], w_ref[...],
                         preferred_element_type=jnp.float32)


def kernel(hidden_states, W):
    M, K = hidden_states.shape
    K2, N = W.shape
    assert K == K2 and M % _BM == 0
    grid = (M // _BM,)
    return pl.pallas_call(
        _router_matmul_kernel,
        grid=grid,
        in_specs=[
            pl.BlockSpec((_BM, K), lambda i: (i, 0)),
            pl.BlockSpec((K, N), lambda i: (0, 0)),
        ],
        out_specs=pl.BlockSpec((_BM, N), lambda i: (i, 0)),
        out_shape=jax.ShapeDtypeStruct((M, N), jnp.float32),
        compiler_params=pltpu.CompilerParams(
            dimension_semantics=("parallel",),
        ),
    )(hidden_states, W)
